# Initial kernel scaffold; baseline (speedup 1.0000x reference)
import jax
import jax.numpy as jnp
from jax.experimental import pallas as pl


def kernel(x, edge_index, edge_attr, W1, b1, W2, b2, Wu, bu, Wt1, bt1, Wt2, bt2):
    raise NotImplementedError("write your pallas kernel here")


# speedup vs baseline: 1.9109x; 1.9109x over previous
import functools
import jax
import jax.numpy as jnp
from jax.experimental import pallas as pl

N = 10000
E = 320000
D_NODE = 128
D_MSG = 64
H_MSG = 64
H_TEN = 32
D_OUT = 64

EDGE_BLOCK = 3200


def _mlp_body(d_ref, ea_ref, w1x_ref, w1e_ref, b1_ref, w2_ref, b2_ref,
              wt1_ref, bt1_ref, wt2_ref, bt2_ref,
              mf_ref, mr_ref, e_ref):
    z1 = jnp.dot(d_ref[...], w1x_ref[...], preferred_element_type=jnp.float32) \
        + jnp.dot(ea_ref[...], w1e_ref[...], preferred_element_type=jnp.float32)
    b1 = b1_ref[...]
    hf = jnp.maximum(z1 + b1, 0.0)
    hr = jnp.maximum(b1 - z1, 0.0)
    w2 = w2_ref[...]
    b2 = b2_ref[...]
    mf = jnp.maximum(jnp.dot(hf, w2, preferred_element_type=jnp.float32) + b2, 0.0)
    mr = jnp.maximum(jnp.dot(hr, w2, preferred_element_type=jnp.float32) + b2, 0.0)
    mf_ref[...] = mf
    mr_ref[...] = mr
    t = jnp.maximum(jnp.dot(mf + mr, wt1_ref[...],
                            preferred_element_type=jnp.float32) + bt1_ref[...], 0.0)
    e_ref[...] = jnp.dot(t, wt2_ref[...], preferred_element_type=jnp.float32) \
        + bt2_ref[...]


def _edge_mlp(d, ea, w1x, w1e, b1, w2, b2, wt1, bt1, wt2, bt2):
    nb = E // EDGE_BLOCK
    full = lambda s: pl.BlockSpec(s, lambda i: (0,) * len(s))
    return pl.pallas_call(
        _mlp_body,
        grid=(nb,),
        in_specs=[
            pl.BlockSpec((EDGE_BLOCK, D_NODE), lambda i: (i, 0)),
            pl.BlockSpec((EDGE_BLOCK, 4), lambda i: (i, 0)),
            full((D_NODE, H_MSG)),
            full((4, H_MSG)),
            full((H_MSG,)),
            full((H_MSG, D_MSG)),
            full((D_MSG,)),
            full((D_MSG, H_TEN)),
            full((H_TEN,)),
            full((H_TEN, 1)),
            full((1,)),
        ],
        out_specs=[
            pl.BlockSpec((EDGE_BLOCK, D_MSG), lambda i: (i, 0)),
            pl.BlockSpec((EDGE_BLOCK, D_MSG), lambda i: (i, 0)),
            pl.BlockSpec((EDGE_BLOCK, 1), lambda i: (i, 0)),
        ],
        out_shape=[
            jax.ShapeDtypeStruct((E, D_MSG), jnp.float32),
            jax.ShapeDtypeStruct((E, D_MSG), jnp.float32),
            jax.ShapeDtypeStruct((E, 1), jnp.float32),
        ],
    )(d, ea, w1x, w1e, b1, w2, b2, wt1, bt1, wt2, bt2)


def kernel(x, edge_index, edge_attr, W1, b1, W2, b2, Wu, bu, Wt1, bt1, Wt2, bt2):
    src = edge_index[0]
    tgt = edge_index[1]
    d = jnp.take(x, tgt, axis=0) - jnp.take(x, src, axis=0)
    mf, mr, e = _edge_mlp(d, edge_attr, W1[:D_NODE], W1[D_NODE:], b1, W2, b2,
                          Wt1, bt1, Wt2, bt2)
    agg = jax.ops.segment_sum(mf, tgt, num_segments=N) + \
        jax.ops.segment_sum(mr, src, num_segments=N)
    ones = jnp.ones((E,), jnp.float32)
    cnt = jax.ops.segment_sum(ones, tgt, num_segments=N) + \
        jax.ops.segment_sum(ones, src, num_segments=N)
    mean = agg / jnp.maximum(cnt, 1.0)[:, None]
    x_out = jnp.dot(x, Wu[:D_NODE], preferred_element_type=jnp.float32) \
        + jnp.dot(mean, Wu[D_NODE:], preferred_element_type=jnp.float32) + bu
    return (x_out, e.reshape(-1))


# speedup vs baseline: 3.1283x; 1.6371x over previous
import functools
import jax
import jax.numpy as jnp
from jax import lax
from jax.experimental import pallas as pl
from jax.experimental.pallas import tpu as pltpu
from jax.experimental.pallas import tpu_sc as plsc

N = 10000
E = 320000
D_NODE = 128
D_MSG = 64
H_MSG = 64
H_TEN = 32
D_OUT = 64

EDGE_BLOCK = 3200

NC = 2
NS = 16
NW = NC * NS
GC = 256
GSUB = GC // 128
NCHUNK = E // GC


def _gather_body(x_ref, tgt_ref, src_ref, gt_ref, gs_ref,
                 it_buf, is_buf, bt, bs):
    w = lax.axis_index("s") * NC + lax.axis_index("c")
    nk = (NCHUNK - w + NW - 1) // NW

    def chunk(k, _):
        c = w + k * NW
        base = c * GC
        for j in range(GSUB):
            pltpu.sync_copy(tgt_ref.at[pl.ds(base + 128 * j, 128)], it_buf.at[j])
            pltpu.sync_copy(src_ref.at[pl.ds(base + 128 * j, 128)], is_buf.at[j])
        for j in range(GSUB):
            pltpu.sync_copy(x_ref.at[it_buf.at[j]], bt.at[pl.ds(128 * j, 128)])
            pltpu.sync_copy(x_ref.at[is_buf.at[j]], bs.at[pl.ds(128 * j, 128)])
        pltpu.sync_copy(bt, gt_ref.at[pl.ds(base, GC)])
        pltpu.sync_copy(bs, gs_ref.at[pl.ds(base, GC)])
        return 0

    lax.fori_loop(0, nk, chunk, 0)


def _sc_gather(x, tgt, src):
    mesh = plsc.VectorSubcoreMesh(core_axis_name="c", subcore_axis_name="s",
                                  num_cores=NC, num_subcores=NS)
    f = pl.kernel(
        _gather_body,
        out_type=[
            jax.ShapeDtypeStruct((E, D_NODE), jnp.float32),
            jax.ShapeDtypeStruct((E, D_NODE), jnp.float32),
        ],
        mesh=mesh,
        scratch_types=[
            pltpu.VMEM((GSUB, 128), jnp.int32),
            pltpu.VMEM((GSUB, 128), jnp.int32),
            pltpu.VMEM((GC, D_NODE), jnp.float32),
            pltpu.VMEM((GC, D_NODE), jnp.float32),
        ],
    )
    return f(x, tgt, src)


def _mlp_body(gt_ref, gs_ref, ea_ref, w1x_ref, w1e_ref, b1_ref, w2_ref, b2_ref,
              wt1_ref, bt1_ref, wt2_ref, bt2_ref,
              mf_ref, mr_ref, e_ref):
    d = gt_ref[...] - gs_ref[...]
    z1 = jnp.dot(d, w1x_ref[...], preferred_element_type=jnp.float32) \
        + jnp.dot(ea_ref[...], w1e_ref[...], preferred_element_type=jnp.float32)
    b1 = b1_ref[...]
    hf = jnp.maximum(z1 + b1, 0.0)
    hr = jnp.maximum(b1 - z1, 0.0)
    w2 = w2_ref[...]
    b2 = b2_ref[...]
    mf = jnp.maximum(jnp.dot(hf, w2, preferred_element_type=jnp.float32) + b2, 0.0)
    mr = jnp.maximum(jnp.dot(hr, w2, preferred_element_type=jnp.float32) + b2, 0.0)
    mf_ref[...] = mf
    mr_ref[...] = mr
    t = jnp.maximum(jnp.dot(mf + mr, wt1_ref[...],
                            preferred_element_type=jnp.float32) + bt1_ref[...], 0.0)
    e_ref[...] = jnp.dot(t, wt2_ref[...], preferred_element_type=jnp.float32) \
        + bt2_ref[...]


def _edge_mlp(gt, gs, ea, w1x, w1e, b1, w2, b2, wt1, bt1, wt2, bt2):
    nb = E // EDGE_BLOCK
    full = lambda s: pl.BlockSpec(s, lambda i: (0,) * len(s))
    return pl.pallas_call(
        _mlp_body,
        grid=(nb,),
        in_specs=[
            pl.BlockSpec((EDGE_BLOCK, D_NODE), lambda i: (i, 0)),
            pl.BlockSpec((EDGE_BLOCK, D_NODE), lambda i: (i, 0)),
            pl.BlockSpec((EDGE_BLOCK, 4), lambda i: (i, 0)),
            full((D_NODE, H_MSG)),
            full((4, H_MSG)),
            full((H_MSG,)),
            full((H_MSG, D_MSG)),
            full((D_MSG,)),
            full((D_MSG, H_TEN)),
            full((H_TEN,)),
            full((H_TEN, 1)),
            full((1,)),
        ],
        out_specs=[
            pl.BlockSpec((EDGE_BLOCK, D_MSG), lambda i: (i, 0)),
            pl.BlockSpec((EDGE_BLOCK, D_MSG), lambda i: (i, 0)),
            pl.BlockSpec((EDGE_BLOCK, 1), lambda i: (i, 0)),
        ],
        out_shape=[
            jax.ShapeDtypeStruct((E, D_MSG), jnp.float32),
            jax.ShapeDtypeStruct((E, D_MSG), jnp.float32),
            jax.ShapeDtypeStruct((E, 1), jnp.float32),
        ],
    )(gt, gs, ea, w1x, w1e, b1, w2, b2, wt1, bt1, wt2, bt2)


def kernel(x, edge_index, edge_attr, W1, b1, W2, b2, Wu, bu, Wt1, bt1, Wt2, bt2):
    src = edge_index[0]
    tgt = edge_index[1]
    gt, gs = _sc_gather(x, tgt, src)
    mf, mr, e = _edge_mlp(gt, gs, edge_attr, W1[:D_NODE], W1[D_NODE:], b1,
                          W2, b2, Wt1, bt1, Wt2, bt2)
    agg = jax.ops.segment_sum(mf, tgt, num_segments=N) + \
        jax.ops.segment_sum(mr, src, num_segments=N)
    ones = jnp.ones((E,), jnp.float32)
    cnt = jax.ops.segment_sum(ones, tgt, num_segments=N) + \
        jax.ops.segment_sum(ones, src, num_segments=N)
    mean = agg / jnp.maximum(cnt, 1.0)[:, None]
    x_out = jnp.dot(x, Wu[:D_NODE], preferred_element_type=jnp.float32) \
        + jnp.dot(mean, Wu[D_NODE:], preferred_element_type=jnp.float32) + bu
    return (x_out, e.reshape(-1))
